# baseline (speedup 1.0000x reference)
import jax
import jax.numpy as jnp
from jax import lax
from jax.experimental import pallas as pl
from jax.experimental.pallas import tpu as pltpu
from jax.experimental.pallas import tpu_sc as plsc

N = 10000
E = 320000
G = 8
DN = 128
DE = 16
DG = 32
HE = 32
HN = 32

NC = 2
NS = 16
NW = NC * NS
EPW = E // NW
CH = 80
NCH = EPW // CH
NB = 1000
EB = 4000

_f32 = jnp.float32
_i32 = jnp.int32


def _k1_body(x_ref, b_ref, wst_ref, wdt_ref, wgt_ref, gf_ref, be1_ref,
             a_ref, bb_ref):
    x = x_ref[...]
    oh = (b_ref[...] == lax.broadcasted_iota(_i32, (1, G), 1)).astype(_f32)
    ga = jnp.dot(gf_ref[...], wgt_ref[...], preferred_element_type=_f32)
    a = jnp.dot(x, wst_ref[...], preferred_element_type=_f32)
    a = a + jnp.dot(oh, ga, preferred_element_type=_f32) + be1_ref[...]
    a_ref[...] = a
    bb_ref[...] = jnp.dot(x, wdt_ref[...], preferred_element_type=_f32)


def _make_tables(x, batch2d, WsT, WdT, WgT, gf, be1):
    nblk = N // NB
    return pl.pallas_call(
        _k1_body,
        grid=(nblk,),
        in_specs=[
            pl.BlockSpec((NB, DN), lambda i: (i, 0)),
            pl.BlockSpec((NB, 1), lambda i: (i, 0)),
            pl.BlockSpec((DN, HE), lambda i: (0, 0)),
            pl.BlockSpec((DN, HE), lambda i: (0, 0)),
            pl.BlockSpec((DG, HE), lambda i: (0, 0)),
            pl.BlockSpec((G, DG), lambda i: (0, 0)),
            pl.BlockSpec((1, HE), lambda i: (0, 0)),
        ],
        out_specs=[
            pl.BlockSpec((NB, HE), lambda i: (i, 0)),
            pl.BlockSpec((NB, HE), lambda i: (i, 0)),
        ],
        out_shape=[
            jax.ShapeDtypeStruct((N, HE), _f32),
            jax.ShapeDtypeStruct((N, HE), _f32),
        ],
    )(x, batch2d, WsT, WdT, WgT, gf, be1)


def _sc_gather_body(a_hbm, b_hbm, src_hbm, dst_hbm, sa_hbm, sb_hbm,
                    idxs, idxd, ra0, ra1, rb0, rb1,
                    sga0, sga1, sgb0, sgb1, ssa0, ssa1, ssb0, ssb1):
    wid = lax.axis_index("c") * NS + lax.axis_index("s")
    row0 = wid * NCH
    base = wid * EPW
    pltpu.sync_copy(src_hbm.at[pl.ds(row0, NCH)], idxs)
    pltpu.sync_copy(dst_hbm.at[pl.ds(row0, NCH)], idxd)

    ra = (ra0, ra1)
    rb = (rb0, rb1)
    sga = (sga0, sga1)
    sgb = (sgb0, sgb1)
    ssa = (ssa0, ssa1)
    ssb = (ssb0, ssb1)

    def issue_gather(j, b):
        pltpu.async_copy(a_hbm.at[idxs.at[j]], ra[b], sga[b])
        pltpu.async_copy(b_hbm.at[idxd.at[j]], rb[b], sgb[b])

    def wait_gather(b):
        pltpu.make_async_copy(a_hbm.at[idxs.at[0]], ra[b], sga[b]).wait()
        pltpu.make_async_copy(b_hbm.at[idxd.at[0]], rb[b], sgb[b]).wait()

    def issue_store(j, b):
        off = base + j * CH
        pltpu.async_copy(ra[b], sa_hbm.at[pl.ds(off, CH)], ssa[b])
        pltpu.async_copy(rb[b], sb_hbm.at[pl.ds(off, CH)], ssb[b])

    def wait_store(b):
        pltpu.make_async_copy(ra[b], sa_hbm.at[pl.ds(base, CH)], ssa[b]).wait()
        pltpu.make_async_copy(rb[b], sb_hbm.at[pl.ds(base, CH)], ssb[b]).wait()

    issue_gather(0, 0)
    issue_gather(1, 1)
    wait_gather(0)
    issue_store(0, 0)

    def body(m, carry):
        j1 = 2 * m + 1
        wait_store(0)
        issue_gather(j1 + 1, 0)
        wait_gather(1)
        issue_store(j1, 1)
        j2 = j1 + 1
        wait_store(1)
        issue_gather(j2 + 1, 1)
        wait_gather(0)
        issue_store(j2, 0)
        return carry

    lax.fori_loop(0, (NCH - 3) // 2, body, 0)
    wait_store(0)
    issue_gather(NCH - 1, 0)
    wait_gather(1)
    issue_store(NCH - 2, 1)
    wait_store(1)
    wait_gather(0)
    issue_store(NCH - 1, 0)
    wait_store(0)


def _gather_tables(A, B, src2d, dst2d):
    mesh = plsc.VectorSubcoreMesh(core_axis_name="c", subcore_axis_name="s",
                                  num_cores=NC, num_subcores=NS)
    fn = pl.kernel(
        _sc_gather_body,
        out_type=[
            jax.ShapeDtypeStruct((E, HE), _f32),
            jax.ShapeDtypeStruct((E, HE), _f32),
        ],
        mesh=mesh,
        scratch_types=[
            pltpu.VMEM((NCH, CH), _i32),
            pltpu.VMEM((NCH, CH), _i32),
            pltpu.VMEM((CH, HE), _f32),
            pltpu.VMEM((CH, HE), _f32),
            pltpu.VMEM((CH, HE), _f32),
            pltpu.VMEM((CH, HE), _f32),
        ] + [pltpu.SemaphoreType.DMA] * 8,
        compiler_params=pltpu.CompilerParams(use_tc_tiling_on_sc=False),
    )
    return fn(A, B, src2d, dst2d)


def _k3_body(sa_ref, sb_ref, ea_ref, wc_ref, w2_ref, b2_ref, w3_ref, b3_ref,
             he_ref):
    s = sa_ref[...] + sb_ref[...]
    c = jnp.dot(ea_ref[...], wc_ref[...], preferred_element_type=_f32)
    h1 = jax.nn.softplus(s + c)
    h2 = jax.nn.softplus(
        jnp.dot(h1, w2_ref[...], preferred_element_type=_f32) + b2_ref[...])
    he_ref[...] = (
        jnp.dot(h2, w3_ref[...], preferred_element_type=_f32) + b3_ref[...])


def _edge_mlp(sa, sb, ea, WcT, W2T, b2, W3T, b3):
    nblk = E // EB
    return pl.pallas_call(
        _k3_body,
        grid=(nblk,),
        in_specs=[
            pl.BlockSpec((EB, HE), lambda i: (i, 0)),
            pl.BlockSpec((EB, HE), lambda i: (i, 0)),
            pl.BlockSpec((EB, DE), lambda i: (i, 0)),
            pl.BlockSpec((DE, HE), lambda i: (0, 0)),
            pl.BlockSpec((HE, HE), lambda i: (0, 0)),
            pl.BlockSpec((1, HE), lambda i: (0, 0)),
            pl.BlockSpec((HE, HE), lambda i: (0, 0)),
            pl.BlockSpec((1, HE), lambda i: (0, 0)),
        ],
        out_specs=pl.BlockSpec((EB, HE), lambda i: (i, 0)),
        out_shape=jax.ShapeDtypeStruct((E, HE), _f32),
    )(sa, sb, ea, WcT, W2T, b2, W3T, b3)


def _sc_scatter_body(he_hbm, src_hbm, dst_hbm, z32_hbm, z16_hbm, ones_hbm,
                     sum_d_hbm, cnt_d_hbm, sum_s_hbm, cnt_s_hbm,
                     idxs, idxd, r0, r1, ones_v,
                     acc_d, acc_s, cnt_d, cnt_s,
                     sl0, sl1, sd0, sd1, ss0, ss1, scd0, scd1, scs0, scs1):
    cid = lax.axis_index("c")
    tid = lax.axis_index("s")
    wid = cid * NS + tid
    row0 = wid * NCH
    base = wid * EPW
    npt = N // NS

    sl = pl.ds(tid * npt, npt)
    pltpu.sync_copy(z32_hbm.at[sl], acc_d.at[sl])
    pltpu.sync_copy(z32_hbm.at[sl], acc_s.at[sl])
    pltpu.sync_copy(z16_hbm.at[sl], cnt_d.at[sl])
    pltpu.sync_copy(z16_hbm.at[sl], cnt_s.at[sl])
    pltpu.sync_copy(ones_hbm, ones_v)
    pltpu.sync_copy(src_hbm.at[pl.ds(row0, NCH)], idxs)
    pltpu.sync_copy(dst_hbm.at[pl.ds(row0, NCH)], idxd)
    plsc.subcore_barrier()

    r = (r0, r1)
    slm = (sl0, sl1)
    sd = (sd0, sd1)
    ss = (ss0, ss1)
    scd = (scd0, scd1)
    scs = (scs0, scs1)

    def issue_load(j, b):
        pltpu.async_copy(he_hbm.at[pl.ds(base + j * CH, CH)], r[b], slm[b])

    def wait_load(b):
        pltpu.make_async_copy(he_hbm.at[pl.ds(base, CH)], r[b], slm[b]).wait()

    def issue_scatter(j, b):
        pltpu.async_copy(r[b], acc_d.at[idxd.at[j]], sd[b], add=True)
        pltpu.async_copy(r[b], acc_s.at[idxs.at[j]], ss[b], add=True)
        pltpu.async_copy(ones_v, cnt_d.at[idxd.at[j]], scd[b], add=True)
        pltpu.async_copy(ones_v, cnt_s.at[idxs.at[j]], scs[b], add=True)

    def wait_scatter(b):
        pltpu.make_async_copy(r[b], acc_d.at[idxd.at[0]], sd[b]).wait()
        pltpu.make_async_copy(r[b], acc_s.at[idxs.at[0]], ss[b]).wait()
        pltpu.make_async_copy(ones_v, cnt_d.at[idxd.at[0]], scd[b]).wait()
        pltpu.make_async_copy(ones_v, cnt_s.at[idxs.at[0]], scs[b]).wait()

    issue_load(0, 0)
    issue_load(1, 1)
    wait_load(0)
    issue_scatter(0, 0)

    def body(m, carry):
        j1 = 2 * m + 1
        wait_scatter(0)
        issue_load(j1 + 1, 0)
        wait_load(1)
        issue_scatter(j1, 1)
        j2 = j1 + 1
        wait_scatter(1)
        issue_load(j2 + 1, 1)
        wait_load(0)
        issue_scatter(j2, 0)
        return carry

    lax.fori_loop(0, (NCH - 3) // 2, body, 0)
    wait_scatter(0)
    issue_load(NCH - 1, 0)
    wait_load(1)
    issue_scatter(NCH - 2, 1)
    wait_scatter(1)
    wait_load(0)
    issue_scatter(NCH - 1, 0)
    wait_scatter(0)
    plsc.subcore_barrier()

    pltpu.sync_copy(acc_d.at[sl], sum_d_hbm.at[cid].at[sl])
    pltpu.sync_copy(acc_s.at[sl], sum_s_hbm.at[cid].at[sl])
    pltpu.sync_copy(cnt_d.at[sl], cnt_d_hbm.at[cid].at[sl])
    pltpu.sync_copy(cnt_s.at[sl], cnt_s_hbm.at[cid].at[sl])


def _scatter_edges(he, src2d, dst2d, z32, z16, ones16):
    mesh = plsc.VectorSubcoreMesh(core_axis_name="c", subcore_axis_name="s",
                                  num_cores=NC, num_subcores=NS)
    fn = pl.kernel(
        _sc_scatter_body,
        out_type=[
            jax.ShapeDtypeStruct((NC, N, HE), _f32),
            jax.ShapeDtypeStruct((NC, N, 16), _f32),
            jax.ShapeDtypeStruct((NC, N, HE), _f32),
            jax.ShapeDtypeStruct((NC, N, 16), _f32),
        ],
        mesh=mesh,
        scratch_types=[
            pltpu.VMEM((NCH, CH), _i32),
            pltpu.VMEM((NCH, CH), _i32),
            pltpu.VMEM((CH, HE), _f32),
            pltpu.VMEM((CH, HE), _f32),
            pltpu.VMEM((CH, 16), _f32),
            pltpu.VMEM_SHARED((N, HE), _f32),
            pltpu.VMEM_SHARED((N, HE), _f32),
            pltpu.VMEM_SHARED((N, 16), _f32),
            pltpu.VMEM_SHARED((N, 16), _f32),
        ] + [pltpu.SemaphoreType.DMA] * 10,
        compiler_params=pltpu.CompilerParams(use_tc_tiling_on_sc=False),
    )
    return fn(he, src2d, dst2d, z32, z16, ones16)


def _k5_body(x_ref, b_ref, sd_ref, cd_ref, ss_ref, cs_ref,
             wn1a_ref, wn1b_ref, wgt_ref, gf_ref, bn1_ref,
             wn2_ref, bn2_ref, wn3_ref, bn3_ref,
             wg1_ref, bg1_ref, wg2_ref, bg2_ref, wg3_ref, bg3_ref,
             hn_ref, hu_ref,
             acc_es, acc_ec, acc_ns, acc_nc):
    i = pl.program_id(0)

    @pl.when(i == 0)
    def _():
        acc_es[...] = jnp.zeros_like(acc_es)
        acc_ec[...] = jnp.zeros_like(acc_ec)
        acc_ns[...] = jnp.zeros_like(acc_ns)
        acc_nc[...] = jnp.zeros_like(acc_nc)

    x = x_ref[...]
    oh = (b_ref[...] == lax.broadcasted_iota(_i32, (1, G), 1)).astype(_f32)
    sd = sd_ref[0] + sd_ref[1]
    cd = cd_ref[0][:, 0:1] + cd_ref[1][:, 0:1]
    e_mean = sd / jnp.maximum(cd, 1.0)
    gn = jnp.dot(gf_ref[...], wgt_ref[...], preferred_element_type=_f32)
    h1 = jnp.dot(x, wn1a_ref[...], preferred_element_type=_f32)
    h1 = h1 + jnp.dot(e_mean, wn1b_ref[...], preferred_element_type=_f32)
    h1 = h1 + jnp.dot(oh, gn, preferred_element_type=_f32) + bn1_ref[...]
    h1 = jax.nn.softplus(h1)
    h2 = jax.nn.softplus(
        jnp.dot(h1, wn2_ref[...], preferred_element_type=_f32) + bn2_ref[...])
    hn = jnp.dot(h2, wn3_ref[...], preferred_element_type=_f32) + bn3_ref[...]
    hn_ref[...] = hn

    ss = ss_ref[0] + ss_ref[1]
    cs = cs_ref[0][:, 0:1] + cs_ref[1][:, 0:1]
    dn = (((0,), (0,)), ((), ()))
    acc_es[...] += lax.dot_general(oh, ss, dn, preferred_element_type=_f32)
    acc_ec[...] += lax.dot_general(oh, cs, dn, preferred_element_type=_f32)
    acc_ns[...] += lax.dot_general(oh, hn, dn, preferred_element_type=_f32)
    acc_nc[...] += lax.dot_general(oh, jnp.ones_like(cs), dn,
                                   preferred_element_type=_f32)

    e_mg = acc_es[...] / jnp.maximum(acc_ec[...], 1.0)
    n_mg = acc_ns[...] / jnp.maximum(acc_nc[...], 1.0)
    gin = jnp.concatenate([e_mg, n_mg, gf_ref[...]], axis=1)
    g1 = jax.nn.softplus(
        jnp.dot(gin, wg1_ref[...], preferred_element_type=_f32) + bg1_ref[...])
    g2 = jax.nn.softplus(
        jnp.dot(g1, wg2_ref[...], preferred_element_type=_f32) + bg2_ref[...])
    hu_ref[...] = (
        jnp.dot(g2, wg3_ref[...], preferred_element_type=_f32) + bg3_ref[...])


def _node_global(x, batch2d, sum_d, cnt_d, sum_s, cnt_s,
                 Wn1aT, Wn1bT, WgnT, gf, bn1, Wn2T, bn2, Wn3T, bn3,
                 Wg1T, bg1, Wg2T, bg2, Wg3T, bg3):
    nblk = N // NB
    const = lambda shape: pl.BlockSpec(shape, lambda i: tuple(0 for _ in shape))
    return pl.pallas_call(
        _k5_body,
        grid=(nblk,),
        in_specs=[
            pl.BlockSpec((NB, DN), lambda i: (i, 0)),
            pl.BlockSpec((NB, 1), lambda i: (i, 0)),
            pl.BlockSpec((NC, NB, HE), lambda i: (0, i, 0)),
            pl.BlockSpec((NC, NB, 16), lambda i: (0, i, 0)),
            pl.BlockSpec((NC, NB, HE), lambda i: (0, i, 0)),
            pl.BlockSpec((NC, NB, 16), lambda i: (0, i, 0)),
            const((DN, HN)),
            const((HE, HN)),
            const((DG, HN)),
            const((G, DG)),
            const((1, HN)),
            const((HN, HN)),
            const((1, HN)),
            const((HN, HN)),
            const((1, HN)),
            const((HN + HE + DG, DG)),
            const((1, DG)),
            const((DG, DG)),
            const((1, DG)),
            const((DG, DG)),
            const((1, DG)),
        ],
        out_specs=[
            pl.BlockSpec((NB, HN), lambda i: (i, 0)),
            pl.BlockSpec((G, DG), lambda i: (0, 0)),
        ],
        out_shape=[
            jax.ShapeDtypeStruct((N, HN), _f32),
            jax.ShapeDtypeStruct((G, DG), _f32),
        ],
        scratch_shapes=[
            pltpu.VMEM((G, DG), _f32),
            pltpu.VMEM((G, 1), _f32),
            pltpu.VMEM((G, HN), _f32),
            pltpu.VMEM((G, 1), _f32),
        ],
        compiler_params=pltpu.CompilerParams(
            dimension_semantics=("arbitrary",)),
    )(x, batch2d, sum_d, cnt_d, sum_s, cnt_s,
      Wn1aT, Wn1bT, WgnT, gf, bn1, Wn2T, bn2, Wn3T, bn3,
      Wg1T, bg1, Wg2T, bg2, Wg3T, bg3)


def kernel(edge_index, x, edge_attr, global_feats, batch,
           We1, be1, We2, be2, We3, be3,
           Wn1, bn1, Wn2, bn2, Wn3, bn3,
           Wg1, bg1, Wg2, bg2, Wg3, bg3):
    src = edge_index[0].astype(_i32)
    dst = edge_index[1].astype(_i32)
    src2d = src.reshape(E // CH, CH)
    dst2d = dst.reshape(E // CH, CH)
    batch2d = batch.astype(_i32).reshape(N, 1)

    WsT = We1[:, :DN].T
    WdT = We1[:, DN:2 * DN].T
    WcT = We1[:, 2 * DN:2 * DN + DE].T
    WgeT = We1[:, 2 * DN + DE:].T

    A, B = _make_tables(x, batch2d, WsT, WdT, WgeT, global_feats,
                        be1.reshape(1, HE))
    SA, SB = _gather_tables(A, B, src2d, dst2d)
    h_e = _edge_mlp(SA, SB, edge_attr, WcT, We2.T, be2.reshape(1, HE),
                    We3.T, be3.reshape(1, HE))

    z32 = jnp.zeros((N, HE), _f32)
    z16 = jnp.zeros((N, 16), _f32)
    ones16 = jnp.ones((CH, 16), _f32)
    sum_d, cnt_d, sum_s, cnt_s = _scatter_edges(
        h_e, src2d, dst2d, z32, z16, ones16)

    Wn1aT = Wn1[:, :DN].T
    Wn1bT = Wn1[:, DN:DN + HE].T
    WgnT = Wn1[:, DN + HE:].T
    h_n, h_u = _node_global(
        x, batch2d, sum_d, cnt_d, sum_s, cnt_s,
        Wn1aT, Wn1bT, WgnT, global_feats, bn1.reshape(1, HN),
        Wn2.T, bn2.reshape(1, HN), Wn3.T, bn3.reshape(1, HN),
        Wg1.T, bg1.reshape(1, DG), Wg2.T, bg2.reshape(1, DG),
        Wg3.T, bg3.reshape(1, DG))
    return (h_e, h_n, h_u)


# speedup vs baseline: 1.9676x; 1.9676x over previous
import jax
import jax.numpy as jnp
from jax import lax
from jax.experimental import pallas as pl
from jax.experimental.pallas import tpu as pltpu
from jax.experimental.pallas import tpu_sc as plsc

N = 10000
E = 320000
G = 8
DN = 128
DE = 16
DG = 32
HE = 32
HN = 32

NC = 2
NS = 16
NW = NC * NS
EPW = E // NW
CH = 80
NCH = EPW // CH
NB = 1000
EB = 4000

_f32 = jnp.float32
_i32 = jnp.int32


def _k1_body(x_ref, b_ref, wst_ref, wdt_ref, wgt_ref, gf_ref, be1_ref,
             a_ref, bb_ref):
    x = x_ref[...]
    oh = (b_ref[...] == lax.broadcasted_iota(_i32, (1, G), 1)).astype(_f32)
    ga = jnp.dot(gf_ref[...], wgt_ref[...], preferred_element_type=_f32)
    a = jnp.dot(x, wst_ref[...], preferred_element_type=_f32)
    a = a + jnp.dot(oh, ga, preferred_element_type=_f32) + be1_ref[...]
    a_ref[...] = a
    bb_ref[...] = jnp.dot(x, wdt_ref[...], preferred_element_type=_f32)


def _make_tables(x, batch2d, WsT, WdT, WgT, gf, be1):
    nblk = N // NB
    return pl.pallas_call(
        _k1_body,
        grid=(nblk,),
        in_specs=[
            pl.BlockSpec((NB, DN), lambda i: (i, 0)),
            pl.BlockSpec((NB, 1), lambda i: (i, 0)),
            pl.BlockSpec((DN, HE), lambda i: (0, 0)),
            pl.BlockSpec((DN, HE), lambda i: (0, 0)),
            pl.BlockSpec((DG, HE), lambda i: (0, 0)),
            pl.BlockSpec((G, DG), lambda i: (0, 0)),
            pl.BlockSpec((1, HE), lambda i: (0, 0)),
        ],
        out_specs=[
            pl.BlockSpec((NB, HE), lambda i: (i, 0)),
            pl.BlockSpec((NB, HE), lambda i: (i, 0)),
        ],
        out_shape=[
            jax.ShapeDtypeStruct((N, HE), _f32),
            jax.ShapeDtypeStruct((N, HE), _f32),
        ],
    )(x, batch2d, WsT, WdT, WgT, gf, be1)


def _sc_gather_body(a_hbm, b_hbm, src_hbm, dst_hbm, sa_hbm, sb_hbm,
                    idxs, idxd, ra0, ra1, rb0, rb1,
                    sga0, sga1, sgb0, sgb1, ssa0, ssa1, ssb0, ssb1):
    wid = lax.axis_index("c") * NS + lax.axis_index("s")
    row0 = wid * NCH
    base = wid * EPW
    pltpu.sync_copy(src_hbm.at[pl.ds(row0, NCH)], idxs)
    pltpu.sync_copy(dst_hbm.at[pl.ds(row0, NCH)], idxd)

    ra = (ra0, ra1)
    rb = (rb0, rb1)
    sga = (sga0, sga1)
    sgb = (sgb0, sgb1)
    ssa = (ssa0, ssa1)
    ssb = (ssb0, ssb1)

    def issue_gather(j, b):
        pltpu.async_copy(a_hbm.at[idxs.at[j]], ra[b], sga[b])
        pltpu.async_copy(b_hbm.at[idxd.at[j]], rb[b], sgb[b])

    def wait_gather(b):
        pltpu.make_async_copy(a_hbm.at[idxs.at[0]], ra[b], sga[b]).wait()
        pltpu.make_async_copy(b_hbm.at[idxd.at[0]], rb[b], sgb[b]).wait()

    def issue_store(j, b):
        off = base + j * CH
        pltpu.async_copy(ra[b], sa_hbm.at[pl.ds(off, CH)], ssa[b])
        pltpu.async_copy(rb[b], sb_hbm.at[pl.ds(off, CH)], ssb[b])

    def wait_store(b):
        pltpu.make_async_copy(ra[b], sa_hbm.at[pl.ds(base, CH)], ssa[b]).wait()
        pltpu.make_async_copy(rb[b], sb_hbm.at[pl.ds(base, CH)], ssb[b]).wait()

    issue_gather(0, 0)
    issue_gather(1, 1)
    wait_gather(0)
    issue_store(0, 0)

    def body(m, carry):
        j1 = 2 * m + 1
        wait_store(0)
        issue_gather(j1 + 1, 0)
        wait_gather(1)
        issue_store(j1, 1)
        j2 = j1 + 1
        wait_store(1)
        issue_gather(j2 + 1, 1)
        wait_gather(0)
        issue_store(j2, 0)
        return carry

    lax.fori_loop(0, (NCH - 3) // 2, body, 0)
    wait_store(0)
    issue_gather(NCH - 1, 0)
    wait_gather(1)
    issue_store(NCH - 2, 1)
    wait_store(1)
    wait_gather(0)
    issue_store(NCH - 1, 0)
    wait_store(0)


def _gather_tables(A, B, src2d, dst2d):
    mesh = plsc.VectorSubcoreMesh(core_axis_name="c", subcore_axis_name="s",
                                  num_cores=NC, num_subcores=NS)
    fn = pl.kernel(
        _sc_gather_body,
        out_type=[
            jax.ShapeDtypeStruct((E, HE), _f32),
            jax.ShapeDtypeStruct((E, HE), _f32),
        ],
        mesh=mesh,
        scratch_types=[
            pltpu.VMEM((NCH, CH), _i32),
            pltpu.VMEM((NCH, CH), _i32),
            pltpu.VMEM((CH, HE), _f32),
            pltpu.VMEM((CH, HE), _f32),
            pltpu.VMEM((CH, HE), _f32),
            pltpu.VMEM((CH, HE), _f32),
        ] + [pltpu.SemaphoreType.DMA] * 8,
        compiler_params=pltpu.CompilerParams(use_tc_tiling_on_sc=False),
    )
    return fn(A, B, src2d, dst2d)


def _k3_body(sa_ref, sb_ref, ea_ref, wc_ref, w2_ref, b2_ref, w3_ref, b3_ref,
             he_ref):
    s = sa_ref[...] + sb_ref[...]
    c = jnp.dot(ea_ref[...], wc_ref[...], preferred_element_type=_f32)
    h1 = jax.nn.softplus(s + c)
    h2 = jax.nn.softplus(
        jnp.dot(h1, w2_ref[...], preferred_element_type=_f32) + b2_ref[...])
    he_ref[...] = (
        jnp.dot(h2, w3_ref[...], preferred_element_type=_f32) + b3_ref[...])


def _edge_mlp(sa4, sb4, ea4, WC4, W24, b24, W34, b34):
    rows = E // 4
    nblk = rows // EB
    return pl.pallas_call(
        _k3_body,
        grid=(nblk,),
        in_specs=[
            pl.BlockSpec((EB, 128), lambda i: (i, 0)),
            pl.BlockSpec((EB, 128), lambda i: (i, 0)),
            pl.BlockSpec((EB, 64), lambda i: (i, 0)),
            pl.BlockSpec((64, 128), lambda i: (0, 0)),
            pl.BlockSpec((128, 128), lambda i: (0, 0)),
            pl.BlockSpec((1, 128), lambda i: (0, 0)),
            pl.BlockSpec((128, 128), lambda i: (0, 0)),
            pl.BlockSpec((1, 128), lambda i: (0, 0)),
        ],
        out_specs=pl.BlockSpec((EB, 128), lambda i: (i, 0)),
        out_shape=jax.ShapeDtypeStruct((rows, 128), _f32),
    )(sa4, sb4, ea4, WC4, W24, b24, W34, b34)


def _block_diag4(w):
    a, b = w.shape
    out = jnp.zeros((4 * a, 4 * b), w.dtype)
    for k in range(4):
        out = out.at[k * a:(k + 1) * a, k * b:(k + 1) * b].set(w)
    return out


def _sc_scatter_body(he_hbm, src_hbm, dst_hbm, z32_hbm, z16_hbm, ones_hbm,
                     sum_d_hbm, cnt_d_hbm, sum_s_hbm, cnt_s_hbm,
                     idxs, idxd, r0, r1, ones_v,
                     acc_d, acc_s, cnt_d, cnt_s,
                     sl0, sl1, sd0, sd1, ss0, ss1, scd0, scd1, scs0, scs1):
    cid = lax.axis_index("c")
    tid = lax.axis_index("s")
    wid = cid * NS + tid
    row0 = wid * NCH
    base = wid * EPW
    npt = N // NS

    sl = pl.ds(tid * npt, npt)
    pltpu.sync_copy(z32_hbm.at[sl], acc_d.at[sl])
    pltpu.sync_copy(z32_hbm.at[sl], acc_s.at[sl])
    pltpu.sync_copy(z16_hbm.at[sl], cnt_d.at[sl])
    pltpu.sync_copy(z16_hbm.at[sl], cnt_s.at[sl])
    pltpu.sync_copy(ones_hbm, ones_v)
    pltpu.sync_copy(src_hbm.at[pl.ds(row0, NCH)], idxs)
    pltpu.sync_copy(dst_hbm.at[pl.ds(row0, NCH)], idxd)
    plsc.subcore_barrier()

    r = (r0, r1)
    slm = (sl0, sl1)
    sd = (sd0, sd1)
    ss = (ss0, ss1)
    scd = (scd0, scd1)
    scs = (scs0, scs1)

    def issue_load(j, b):
        pltpu.async_copy(he_hbm.at[pl.ds(base + j * CH, CH)], r[b], slm[b])

    def wait_load(b):
        pltpu.make_async_copy(he_hbm.at[pl.ds(base, CH)], r[b], slm[b]).wait()

    def issue_scatter(j, b):
        pltpu.async_copy(r[b], acc_d.at[idxd.at[j]], sd[b], add=True)
        pltpu.async_copy(r[b], acc_s.at[idxs.at[j]], ss[b], add=True)
        pltpu.async_copy(ones_v, cnt_d.at[idxd.at[j]], scd[b], add=True)
        pltpu.async_copy(ones_v, cnt_s.at[idxs.at[j]], scs[b], add=True)

    def wait_scatter(b):
        pltpu.make_async_copy(r[b], acc_d.at[idxd.at[0]], sd[b]).wait()
        pltpu.make_async_copy(r[b], acc_s.at[idxs.at[0]], ss[b]).wait()
        pltpu.make_async_copy(ones_v, cnt_d.at[idxd.at[0]], scd[b]).wait()
        pltpu.make_async_copy(ones_v, cnt_s.at[idxs.at[0]], scs[b]).wait()

    issue_load(0, 0)
    issue_load(1, 1)
    wait_load(0)
    issue_scatter(0, 0)

    def body(m, carry):
        j1 = 2 * m + 1
        wait_scatter(0)
        issue_load(j1 + 1, 0)
        wait_load(1)
        issue_scatter(j1, 1)
        j2 = j1 + 1
        wait_scatter(1)
        issue_load(j2 + 1, 1)
        wait_load(0)
        issue_scatter(j2, 0)
        return carry

    lax.fori_loop(0, (NCH - 3) // 2, body, 0)
    wait_scatter(0)
    issue_load(NCH - 1, 0)
    wait_load(1)
    issue_scatter(NCH - 2, 1)
    wait_scatter(1)
    wait_load(0)
    issue_scatter(NCH - 1, 0)
    wait_scatter(0)
    plsc.subcore_barrier()

    pltpu.sync_copy(acc_d.at[sl], sum_d_hbm.at[cid].at[sl])
    pltpu.sync_copy(acc_s.at[sl], sum_s_hbm.at[cid].at[sl])
    pltpu.sync_copy(cnt_d.at[sl], cnt_d_hbm.at[cid].at[sl])
    pltpu.sync_copy(cnt_s.at[sl], cnt_s_hbm.at[cid].at[sl])


def _scatter_edges(he, src2d, dst2d, z32, z16, ones16):
    mesh = plsc.VectorSubcoreMesh(core_axis_name="c", subcore_axis_name="s",
                                  num_cores=NC, num_subcores=NS)
    fn = pl.kernel(
        _sc_scatter_body,
        out_type=[
            jax.ShapeDtypeStruct((NC, N, HE), _f32),
            jax.ShapeDtypeStruct((NC, N, 16), _f32),
            jax.ShapeDtypeStruct((NC, N, HE), _f32),
            jax.ShapeDtypeStruct((NC, N, 16), _f32),
        ],
        mesh=mesh,
        scratch_types=[
            pltpu.VMEM((NCH, CH), _i32),
            pltpu.VMEM((NCH, CH), _i32),
            pltpu.VMEM((CH, HE), _f32),
            pltpu.VMEM((CH, HE), _f32),
            pltpu.VMEM((CH, 16), _f32),
            pltpu.VMEM_SHARED((N, HE), _f32),
            pltpu.VMEM_SHARED((N, HE), _f32),
            pltpu.VMEM_SHARED((N, 16), _f32),
            pltpu.VMEM_SHARED((N, 16), _f32),
        ] + [pltpu.SemaphoreType.DMA] * 10,
        compiler_params=pltpu.CompilerParams(use_tc_tiling_on_sc=False),
    )
    return fn(he, src2d, dst2d, z32, z16, ones16)


def _k5_body(x_ref, b_ref, sd_ref, cd_ref, ss_ref, cs_ref,
             wn1a_ref, wn1b_ref, wgt_ref, gf_ref, bn1_ref,
             wn2_ref, bn2_ref, wn3_ref, bn3_ref,
             wg1_ref, bg1_ref, wg2_ref, bg2_ref, wg3_ref, bg3_ref,
             hn_ref, hu_ref,
             acc_es, acc_ec, acc_ns, acc_nc):
    i = pl.program_id(0)

    @pl.when(i == 0)
    def _():
        acc_es[...] = jnp.zeros_like(acc_es)
        acc_ec[...] = jnp.zeros_like(acc_ec)
        acc_ns[...] = jnp.zeros_like(acc_ns)
        acc_nc[...] = jnp.zeros_like(acc_nc)

    x = x_ref[...]
    oh = (b_ref[...] == lax.broadcasted_iota(_i32, (1, G), 1)).astype(_f32)
    sd = sd_ref[0] + sd_ref[1]
    cd = cd_ref[0][:, 0:1] + cd_ref[1][:, 0:1]
    e_mean = sd / jnp.maximum(cd, 1.0)
    gn = jnp.dot(gf_ref[...], wgt_ref[...], preferred_element_type=_f32)
    h1 = jnp.dot(x, wn1a_ref[...], preferred_element_type=_f32)
    h1 = h1 + jnp.dot(e_mean, wn1b_ref[...], preferred_element_type=_f32)
    h1 = h1 + jnp.dot(oh, gn, preferred_element_type=_f32) + bn1_ref[...]
    h1 = jax.nn.softplus(h1)
    h2 = jax.nn.softplus(
        jnp.dot(h1, wn2_ref[...], preferred_element_type=_f32) + bn2_ref[...])
    hn = jnp.dot(h2, wn3_ref[...], preferred_element_type=_f32) + bn3_ref[...]
    hn_ref[...] = hn

    ss = ss_ref[0] + ss_ref[1]
    cs = cs_ref[0][:, 0:1] + cs_ref[1][:, 0:1]
    dn = (((0,), (0,)), ((), ()))
    acc_es[...] += lax.dot_general(oh, ss, dn, preferred_element_type=_f32)
    acc_ec[...] += lax.dot_general(oh, cs, dn, preferred_element_type=_f32)
    acc_ns[...] += lax.dot_general(oh, hn, dn, preferred_element_type=_f32)
    acc_nc[...] += lax.dot_general(oh, jnp.ones_like(cs), dn,
                                   preferred_element_type=_f32)

    e_mg = acc_es[...] / jnp.maximum(acc_ec[...], 1.0)
    n_mg = acc_ns[...] / jnp.maximum(acc_nc[...], 1.0)
    gin = jnp.concatenate([e_mg, n_mg, gf_ref[...]], axis=1)
    g1 = jax.nn.softplus(
        jnp.dot(gin, wg1_ref[...], preferred_element_type=_f32) + bg1_ref[...])
    g2 = jax.nn.softplus(
        jnp.dot(g1, wg2_ref[...], preferred_element_type=_f32) + bg2_ref[...])
    hu_ref[...] = (
        jnp.dot(g2, wg3_ref[...], preferred_element_type=_f32) + bg3_ref[...])


def _node_global(x, batch2d, sum_d, cnt_d, sum_s, cnt_s,
                 Wn1aT, Wn1bT, WgnT, gf, bn1, Wn2T, bn2, Wn3T, bn3,
                 Wg1T, bg1, Wg2T, bg2, Wg3T, bg3):
    nblk = N // NB
    const = lambda shape: pl.BlockSpec(shape, lambda i: tuple(0 for _ in shape))
    return pl.pallas_call(
        _k5_body,
        grid=(nblk,),
        in_specs=[
            pl.BlockSpec((NB, DN), lambda i: (i, 0)),
            pl.BlockSpec((NB, 1), lambda i: (i, 0)),
            pl.BlockSpec((NC, NB, HE), lambda i: (0, i, 0)),
            pl.BlockSpec((NC, NB, 16), lambda i: (0, i, 0)),
            pl.BlockSpec((NC, NB, HE), lambda i: (0, i, 0)),
            pl.BlockSpec((NC, NB, 16), lambda i: (0, i, 0)),
            const((DN, HN)),
            const((HE, HN)),
            const((DG, HN)),
            const((G, DG)),
            const((1, HN)),
            const((HN, HN)),
            const((1, HN)),
            const((HN, HN)),
            const((1, HN)),
            const((HN + HE + DG, DG)),
            const((1, DG)),
            const((DG, DG)),
            const((1, DG)),
            const((DG, DG)),
            const((1, DG)),
        ],
        out_specs=[
            pl.BlockSpec((NB, HN), lambda i: (i, 0)),
            pl.BlockSpec((G, DG), lambda i: (0, 0)),
        ],
        out_shape=[
            jax.ShapeDtypeStruct((N, HN), _f32),
            jax.ShapeDtypeStruct((G, DG), _f32),
        ],
        scratch_shapes=[
            pltpu.VMEM((G, DG), _f32),
            pltpu.VMEM((G, 1), _f32),
            pltpu.VMEM((G, HN), _f32),
            pltpu.VMEM((G, 1), _f32),
        ],
        compiler_params=pltpu.CompilerParams(
            dimension_semantics=("arbitrary",)),
    )(x, batch2d, sum_d, cnt_d, sum_s, cnt_s,
      Wn1aT, Wn1bT, WgnT, gf, bn1, Wn2T, bn2, Wn3T, bn3,
      Wg1T, bg1, Wg2T, bg2, Wg3T, bg3)


def kernel(edge_index, x, edge_attr, global_feats, batch,
           We1, be1, We2, be2, We3, be3,
           Wn1, bn1, Wn2, bn2, Wn3, bn3,
           Wg1, bg1, Wg2, bg2, Wg3, bg3):
    src = edge_index[0].astype(_i32)
    dst = edge_index[1].astype(_i32)
    src2d = src.reshape(E // CH, CH)
    dst2d = dst.reshape(E // CH, CH)
    batch2d = batch.astype(_i32).reshape(N, 1)

    WsT = We1[:, :DN].T
    WdT = We1[:, DN:2 * DN].T
    WcT = We1[:, 2 * DN:2 * DN + DE].T
    WgeT = We1[:, 2 * DN + DE:].T

    A, B = _make_tables(x, batch2d, WsT, WdT, WgeT, global_feats,
                        be1.reshape(1, HE))
    SA, SB = _gather_tables(A, B, src2d, dst2d)
    he4 = _edge_mlp(SA.reshape(E // 4, 128), SB.reshape(E // 4, 128),
                    edge_attr.reshape(E // 4, 64),
                    _block_diag4(WcT), _block_diag4(We2.T),
                    jnp.tile(be2, 4).reshape(1, 128),
                    _block_diag4(We3.T), jnp.tile(be3, 4).reshape(1, 128))
    h_e = he4.reshape(E, HE)

    z32 = jnp.zeros((N, HE), _f32)
    z16 = jnp.zeros((N, 16), _f32)
    ones16 = jnp.ones((CH, 16), _f32)
    sum_d, cnt_d, sum_s, cnt_s = _scatter_edges(
        h_e, src2d, dst2d, z32, z16, ones16)

    Wn1aT = Wn1[:, :DN].T
    Wn1bT = Wn1[:, DN:DN + HE].T
    WgnT = Wn1[:, DN + HE:].T
    h_n, h_u = _node_global(
        x, batch2d, sum_d, cnt_d, sum_s, cnt_s,
        Wn1aT, Wn1bT, WgnT, global_feats, bn1.reshape(1, HN),
        Wn2.T, bn2.reshape(1, HN), Wn3.T, bn3.reshape(1, HN),
        Wg1.T, bg1.reshape(1, DG), Wg2.T, bg2.reshape(1, DG),
        Wg3.T, bg3.reshape(1, DG))
    return (h_e, h_n, h_u)


# speedup vs baseline: 2.0521x; 1.0429x over previous
import jax
import jax.numpy as jnp
from jax import lax
from jax.experimental import pallas as pl
from jax.experimental.pallas import tpu as pltpu
from jax.experimental.pallas import tpu_sc as plsc

N = 10000
E = 320000
G = 8
DN = 128
DE = 16
DG = 32
HE = 32
HN = 32

NC = 2
NS = 16
NW = NC * NS
EPW = E // NW
CH = 80
NCH = EPW // CH
NB = 1000
EB = 4000

_f32 = jnp.float32
_i32 = jnp.int32


def _k1_body(x_ref, b_ref, wst_ref, wdt_ref, wgt_ref, gf_ref, be1_ref,
             a_ref, bb_ref):
    x = x_ref[...]
    oh = (b_ref[...] == lax.broadcasted_iota(_i32, (1, G), 1)).astype(_f32)
    ga = jnp.dot(gf_ref[...], wgt_ref[...], preferred_element_type=_f32)
    a = jnp.dot(x, wst_ref[...], preferred_element_type=_f32)
    a = a + jnp.dot(oh, ga, preferred_element_type=_f32) + be1_ref[...]
    a_ref[...] = a
    bb_ref[...] = jnp.dot(x, wdt_ref[...], preferred_element_type=_f32)


def _make_tables(x, batch2d, WsT, WdT, WgT, gf, be1):
    nblk = N // NB
    return pl.pallas_call(
        _k1_body,
        grid=(nblk,),
        in_specs=[
            pl.BlockSpec((NB, DN), lambda i: (i, 0)),
            pl.BlockSpec((NB, 1), lambda i: (i, 0)),
            pl.BlockSpec((DN, HE), lambda i: (0, 0)),
            pl.BlockSpec((DN, HE), lambda i: (0, 0)),
            pl.BlockSpec((DG, HE), lambda i: (0, 0)),
            pl.BlockSpec((G, DG), lambda i: (0, 0)),
            pl.BlockSpec((1, HE), lambda i: (0, 0)),
        ],
        out_specs=[
            pl.BlockSpec((NB, HE), lambda i: (i, 0)),
            pl.BlockSpec((NB, HE), lambda i: (i, 0)),
        ],
        out_shape=[
            jax.ShapeDtypeStruct((N, HE), _f32),
            jax.ShapeDtypeStruct((N, HE), _f32),
        ],
    )(x, batch2d, WsT, WdT, WgT, gf, be1)


CPR = CH * HE // 128


def _sc_gather_body(a_hbm, b_hbm, src_hbm, dst_hbm, s4_hbm,
                    idxs, idxd, ra0, ra1, rb0, rb1, ob0, ob1,
                    sga0, sga1, sgb0, sgb1, sst0, sst1):
    wid = lax.axis_index("c") * NS + lax.axis_index("s")
    row0 = wid * NCH
    pltpu.sync_copy(src_hbm.at[pl.ds(row0, NCH)], idxs)
    pltpu.sync_copy(dst_hbm.at[pl.ds(row0, NCH)], idxd)

    ra = (ra0, ra1)
    rb = (rb0, rb1)
    ob = (ob0, ob1)
    sga = (sga0, sga1)
    sgb = (sgb0, sgb1)
    sst = (sst0, sst1)

    def issue_gather(j, b):
        pltpu.async_copy(a_hbm.at[idxs.at[j]], ra[b], sga[b])
        pltpu.async_copy(b_hbm.at[idxd.at[j]], rb[b], sgb[b])

    def wait_gather(b):
        pltpu.make_async_copy(a_hbm.at[idxs.at[0]], ra[b], sga[b]).wait()
        pltpu.make_async_copy(b_hbm.at[idxd.at[0]], rb[b], sgb[b]).wait()

    def add_repack(b):
        def rows(p, carry):
            for m in range(4):
                for h in range(2):
                    v = (ra[b][4 * p + m, pl.ds(16 * h, 16)]
                         + rb[b][4 * p + m, pl.ds(16 * h, 16)])
                    ob[b][p, pl.ds(32 * m + 16 * h, 16)] = v
            return carry
        lax.fori_loop(0, CPR, rows, 0)

    def issue_store(j, b):
        off = wid * (NCH * CPR) + j * CPR
        pltpu.async_copy(ob[b], s4_hbm.at[pl.ds(off, CPR)], sst[b])

    def wait_store(b):
        pltpu.make_async_copy(ob[b], s4_hbm.at[pl.ds(0, CPR)], sst[b]).wait()

    def step(j, b, first, last):
        wait_gather(b)
        if not first:
            wait_store(b)
        add_repack(b)
        issue_store(j, b)
        if not last:
            issue_gather(j + 2, b)

    issue_gather(0, 0)
    issue_gather(1, 1)
    step(0, 0, True, False)
    step(1, 1, True, False)

    def body(m, carry):
        step(2 * m + 2, 0, False, False)
        step(2 * m + 3, 1, False, False)
        return carry

    lax.fori_loop(0, (NCH - 5) // 2, body, 0)
    step(NCH - 3, 0, False, False)
    step(NCH - 2, 1, False, True)
    step(NCH - 1, 0, False, True)
    wait_store(1)
    wait_store(0)


def _gather_tables(A, B, src2d, dst2d):
    mesh = plsc.VectorSubcoreMesh(core_axis_name="c", subcore_axis_name="s",
                                  num_cores=NC, num_subcores=NS)
    fn = pl.kernel(
        _sc_gather_body,
        out_type=jax.ShapeDtypeStruct((E * HE // 128, 128), _f32),
        mesh=mesh,
        scratch_types=[
            pltpu.VMEM((NCH, CH), _i32),
            pltpu.VMEM((NCH, CH), _i32),
            pltpu.VMEM((CH, HE), _f32),
            pltpu.VMEM((CH, HE), _f32),
            pltpu.VMEM((CH, HE), _f32),
            pltpu.VMEM((CH, HE), _f32),
            pltpu.VMEM((CPR, 128), _f32),
            pltpu.VMEM((CPR, 128), _f32),
        ] + [pltpu.SemaphoreType.DMA] * 6,
        compiler_params=pltpu.CompilerParams(use_tc_tiling_on_sc=False),
    )
    return fn(A, B, src2d, dst2d)


def _k3_body(s4_ref, ea_ref, wc_ref, w2_ref, b2_ref, w3_ref, b3_ref,
             he_ref):
    c = jnp.dot(ea_ref[...], wc_ref[...], preferred_element_type=_f32)
    h1 = jax.nn.softplus(s4_ref[...] + c)
    h2 = jax.nn.softplus(
        jnp.dot(h1, w2_ref[...], preferred_element_type=_f32) + b2_ref[...])
    he_ref[...] = (
        jnp.dot(h2, w3_ref[...], preferred_element_type=_f32) + b3_ref[...])


def _edge_mlp(s4, ea4, WC4, W24, b24, W34, b34):
    rows = E // 4
    nblk = rows // EB
    return pl.pallas_call(
        _k3_body,
        grid=(nblk,),
        in_specs=[
            pl.BlockSpec((EB, 128), lambda i: (i, 0)),
            pl.BlockSpec((EB, 64), lambda i: (i, 0)),
            pl.BlockSpec((64, 128), lambda i: (0, 0)),
            pl.BlockSpec((128, 128), lambda i: (0, 0)),
            pl.BlockSpec((1, 128), lambda i: (0, 0)),
            pl.BlockSpec((128, 128), lambda i: (0, 0)),
            pl.BlockSpec((1, 128), lambda i: (0, 0)),
        ],
        out_specs=pl.BlockSpec((EB, 128), lambda i: (i, 0)),
        out_shape=jax.ShapeDtypeStruct((rows, 128), _f32),
    )(s4, ea4, WC4, W24, b24, W34, b34)


def _block_diag4(w):
    a, b = w.shape
    out = jnp.zeros((4 * a, 4 * b), w.dtype)
    for k in range(4):
        out = out.at[k * a:(k + 1) * a, k * b:(k + 1) * b].set(w)
    return out


def _sc_scatter_body(he_hbm, src_hbm, dst_hbm, z32_hbm, z16_hbm, ones_hbm,
                     sum_d_hbm, cnt_d_hbm, sum_s_hbm, cnt_s_hbm,
                     idxs, idxd, hb0, hb1, r0, r1, ones_v,
                     acc_d, acc_s, cnt_d, cnt_s,
                     sl0, sl1, sd0, sd1, ss0, ss1, scd0, scd1, scs0, scs1):
    cid = lax.axis_index("c")
    tid = lax.axis_index("s")
    wid = cid * NS + tid
    row0 = wid * NCH
    base = wid * EPW
    npt = N // NS

    sl = pl.ds(tid * npt, npt)
    pltpu.sync_copy(z32_hbm.at[sl], acc_d.at[sl])
    pltpu.sync_copy(z32_hbm.at[sl], acc_s.at[sl])
    pltpu.sync_copy(z16_hbm.at[sl], cnt_d.at[sl])
    pltpu.sync_copy(z16_hbm.at[sl], cnt_s.at[sl])
    pltpu.sync_copy(ones_hbm, ones_v)
    pltpu.sync_copy(src_hbm.at[pl.ds(row0, NCH)], idxs)
    pltpu.sync_copy(dst_hbm.at[pl.ds(row0, NCH)], idxd)
    plsc.subcore_barrier()

    hb = (hb0, hb1)
    r = (r0, r1)
    slm = (sl0, sl1)
    sd = (sd0, sd1)
    ss = (ss0, ss1)
    scd = (scd0, scd1)
    scs = (scs0, scs1)

    def issue_load(j, b):
        off = wid * (NCH * CPR) + j * CPR
        pltpu.async_copy(he_hbm.at[pl.ds(off, CPR)], hb[b], slm[b])

    def wait_load(b):
        pltpu.make_async_copy(he_hbm.at[pl.ds(0, CPR)], hb[b], slm[b]).wait()

    def repack(b):
        def rows(p, carry):
            for m in range(4):
                for h in range(2):
                    r[b][4 * p + m, pl.ds(16 * h, 16)] = (
                        hb[b][p, pl.ds(32 * m + 16 * h, 16)])
            return carry
        lax.fori_loop(0, CPR, rows, 0)

    def issue_scatter(j, b):
        pltpu.async_copy(r[b], acc_d.at[idxd.at[j]], sd[b], add=True)
        pltpu.async_copy(r[b], acc_s.at[idxs.at[j]], ss[b], add=True)
        pltpu.async_copy(ones_v, cnt_d.at[idxd.at[j]], scd[b], add=True)
        pltpu.async_copy(ones_v, cnt_s.at[idxs.at[j]], scs[b], add=True)

    def wait_scatter(b):
        pltpu.make_async_copy(r[b], acc_d.at[idxd.at[0]], sd[b]).wait()
        pltpu.make_async_copy(r[b], acc_s.at[idxs.at[0]], ss[b]).wait()
        pltpu.make_async_copy(ones_v, cnt_d.at[idxd.at[0]], scd[b]).wait()
        pltpu.make_async_copy(ones_v, cnt_s.at[idxs.at[0]], scs[b]).wait()

    def step(j, b, first, last):
        wait_load(b)
        if not first:
            wait_scatter(b)
        repack(b)
        issue_scatter(j, b)
        if not last:
            issue_load(j + 2, b)

    issue_load(0, 0)
    issue_load(1, 1)
    step(0, 0, True, False)
    step(1, 1, True, False)

    def body(m, carry):
        step(2 * m + 2, 0, False, False)
        step(2 * m + 3, 1, False, False)
        return carry

    lax.fori_loop(0, (NCH - 5) // 2, body, 0)
    step(NCH - 3, 0, False, False)
    step(NCH - 2, 1, False, True)
    step(NCH - 1, 0, False, True)
    wait_scatter(1)
    wait_scatter(0)
    plsc.subcore_barrier()

    pltpu.sync_copy(acc_d.at[sl], sum_d_hbm.at[cid].at[sl])
    pltpu.sync_copy(acc_s.at[sl], sum_s_hbm.at[cid].at[sl])
    pltpu.sync_copy(cnt_d.at[sl], cnt_d_hbm.at[cid].at[sl])
    pltpu.sync_copy(cnt_s.at[sl], cnt_s_hbm.at[cid].at[sl])


def _scatter_edges(he, src2d, dst2d, z32, z16, ones16):
    mesh = plsc.VectorSubcoreMesh(core_axis_name="c", subcore_axis_name="s",
                                  num_cores=NC, num_subcores=NS)
    fn = pl.kernel(
        _sc_scatter_body,
        out_type=[
            jax.ShapeDtypeStruct((NC, N, HE), _f32),
            jax.ShapeDtypeStruct((NC, N, 16), _f32),
            jax.ShapeDtypeStruct((NC, N, HE), _f32),
            jax.ShapeDtypeStruct((NC, N, 16), _f32),
        ],
        mesh=mesh,
        scratch_types=[
            pltpu.VMEM((NCH, CH), _i32),
            pltpu.VMEM((NCH, CH), _i32),
            pltpu.VMEM((CPR, 128), _f32),
            pltpu.VMEM((CPR, 128), _f32),
            pltpu.VMEM((CH, HE), _f32),
            pltpu.VMEM((CH, HE), _f32),
            pltpu.VMEM((CH, 16), _f32),
            pltpu.VMEM_SHARED((N, HE), _f32),
            pltpu.VMEM_SHARED((N, HE), _f32),
            pltpu.VMEM_SHARED((N, 16), _f32),
            pltpu.VMEM_SHARED((N, 16), _f32),
        ] + [pltpu.SemaphoreType.DMA] * 10,
        compiler_params=pltpu.CompilerParams(use_tc_tiling_on_sc=False),
    )
    return fn(he, src2d, dst2d, z32, z16, ones16)


def _k5_body(x_ref, b_ref, sd_ref, cd_ref, ss_ref, cs_ref,
             wn1a_ref, wn1b_ref, wgt_ref, gf_ref, bn1_ref,
             wn2_ref, bn2_ref, wn3_ref, bn3_ref,
             wg1_ref, bg1_ref, wg2_ref, bg2_ref, wg3_ref, bg3_ref,
             hn_ref, hu_ref,
             acc_es, acc_ec, acc_ns, acc_nc):
    i = pl.program_id(0)

    @pl.when(i == 0)
    def _():
        acc_es[...] = jnp.zeros_like(acc_es)
        acc_ec[...] = jnp.zeros_like(acc_ec)
        acc_ns[...] = jnp.zeros_like(acc_ns)
        acc_nc[...] = jnp.zeros_like(acc_nc)

    x = x_ref[...]
    oh = (b_ref[...] == lax.broadcasted_iota(_i32, (1, G), 1)).astype(_f32)
    sd = sd_ref[0] + sd_ref[1]
    cd = cd_ref[0][:, 0:1] + cd_ref[1][:, 0:1]
    e_mean = sd / jnp.maximum(cd, 1.0)
    gn = jnp.dot(gf_ref[...], wgt_ref[...], preferred_element_type=_f32)
    h1 = jnp.dot(x, wn1a_ref[...], preferred_element_type=_f32)
    h1 = h1 + jnp.dot(e_mean, wn1b_ref[...], preferred_element_type=_f32)
    h1 = h1 + jnp.dot(oh, gn, preferred_element_type=_f32) + bn1_ref[...]
    h1 = jax.nn.softplus(h1)
    h2 = jax.nn.softplus(
        jnp.dot(h1, wn2_ref[...], preferred_element_type=_f32) + bn2_ref[...])
    hn = jnp.dot(h2, wn3_ref[...], preferred_element_type=_f32) + bn3_ref[...]
    hn_ref[...] = hn

    ss = ss_ref[0] + ss_ref[1]
    cs = cs_ref[0][:, 0:1] + cs_ref[1][:, 0:1]
    dn = (((0,), (0,)), ((), ()))
    acc_es[...] += lax.dot_general(oh, ss, dn, preferred_element_type=_f32)
    acc_ec[...] += lax.dot_general(oh, cs, dn, preferred_element_type=_f32)
    acc_ns[...] += lax.dot_general(oh, hn, dn, preferred_element_type=_f32)
    acc_nc[...] += lax.dot_general(oh, jnp.ones_like(cs), dn,
                                   preferred_element_type=_f32)

    e_mg = acc_es[...] / jnp.maximum(acc_ec[...], 1.0)
    n_mg = acc_ns[...] / jnp.maximum(acc_nc[...], 1.0)
    gin = jnp.concatenate([e_mg, n_mg, gf_ref[...]], axis=1)
    g1 = jax.nn.softplus(
        jnp.dot(gin, wg1_ref[...], preferred_element_type=_f32) + bg1_ref[...])
    g2 = jax.nn.softplus(
        jnp.dot(g1, wg2_ref[...], preferred_element_type=_f32) + bg2_ref[...])
    hu_ref[...] = (
        jnp.dot(g2, wg3_ref[...], preferred_element_type=_f32) + bg3_ref[...])


def _node_global(x, batch2d, sum_d, cnt_d, sum_s, cnt_s,
                 Wn1aT, Wn1bT, WgnT, gf, bn1, Wn2T, bn2, Wn3T, bn3,
                 Wg1T, bg1, Wg2T, bg2, Wg3T, bg3):
    nblk = N // NB
    const = lambda shape: pl.BlockSpec(shape, lambda i: tuple(0 for _ in shape))
    return pl.pallas_call(
        _k5_body,
        grid=(nblk,),
        in_specs=[
            pl.BlockSpec((NB, DN), lambda i: (i, 0)),
            pl.BlockSpec((NB, 1), lambda i: (i, 0)),
            pl.BlockSpec((NC, NB, HE), lambda i: (0, i, 0)),
            pl.BlockSpec((NC, NB, 16), lambda i: (0, i, 0)),
            pl.BlockSpec((NC, NB, HE), lambda i: (0, i, 0)),
            pl.BlockSpec((NC, NB, 16), lambda i: (0, i, 0)),
            const((DN, HN)),
            const((HE, HN)),
            const((DG, HN)),
            const((G, DG)),
            const((1, HN)),
            const((HN, HN)),
            const((1, HN)),
            const((HN, HN)),
            const((1, HN)),
            const((HN + HE + DG, DG)),
            const((1, DG)),
            const((DG, DG)),
            const((1, DG)),
            const((DG, DG)),
            const((1, DG)),
        ],
        out_specs=[
            pl.BlockSpec((NB, HN), lambda i: (i, 0)),
            pl.BlockSpec((G, DG), lambda i: (0, 0)),
        ],
        out_shape=[
            jax.ShapeDtypeStruct((N, HN), _f32),
            jax.ShapeDtypeStruct((G, DG), _f32),
        ],
        scratch_shapes=[
            pltpu.VMEM((G, DG), _f32),
            pltpu.VMEM((G, 1), _f32),
            pltpu.VMEM((G, HN), _f32),
            pltpu.VMEM((G, 1), _f32),
        ],
        compiler_params=pltpu.CompilerParams(
            dimension_semantics=("arbitrary",)),
    )(x, batch2d, sum_d, cnt_d, sum_s, cnt_s,
      Wn1aT, Wn1bT, WgnT, gf, bn1, Wn2T, bn2, Wn3T, bn3,
      Wg1T, bg1, Wg2T, bg2, Wg3T, bg3)


def kernel(edge_index, x, edge_attr, global_feats, batch,
           We1, be1, We2, be2, We3, be3,
           Wn1, bn1, Wn2, bn2, Wn3, bn3,
           Wg1, bg1, Wg2, bg2, Wg3, bg3):
    src = edge_index[0].astype(_i32)
    dst = edge_index[1].astype(_i32)
    src2d = src.reshape(E // CH, CH)
    dst2d = dst.reshape(E // CH, CH)
    batch2d = batch.astype(_i32).reshape(N, 1)

    WsT = We1[:, :DN].T
    WdT = We1[:, DN:2 * DN].T
    WcT = We1[:, 2 * DN:2 * DN + DE].T
    WgeT = We1[:, 2 * DN + DE:].T

    A, B = _make_tables(x, batch2d, WsT, WdT, WgeT, global_feats,
                        be1.reshape(1, HE))
    s4 = _gather_tables(A, B, src2d, dst2d)
    he4 = _edge_mlp(s4, edge_attr.reshape(E // 4, 64),
                    _block_diag4(WcT), _block_diag4(We2.T),
                    jnp.tile(be2, 4).reshape(1, 128),
                    _block_diag4(We3.T), jnp.tile(be3, 4).reshape(1, 128))
    h_e = he4.reshape(E, HE)

    z32 = jnp.zeros((N, HE), _f32)
    z16 = jnp.zeros((N, 16), _f32)
    ones16 = jnp.ones((CH, 16), _f32)
    sum_d, cnt_d, sum_s, cnt_s = _scatter_edges(
        he4, src2d, dst2d, z32, z16, ones16)

    Wn1aT = Wn1[:, :DN].T
    Wn1bT = Wn1[:, DN:DN + HE].T
    WgnT = Wn1[:, DN + HE:].T
    h_n, h_u = _node_global(
        x, batch2d, sum_d, cnt_d, sum_s, cnt_s,
        Wn1aT, Wn1bT, WgnT, global_feats, bn1.reshape(1, HN),
        Wn2.T, bn2.reshape(1, HN), Wn3.T, bn3.reshape(1, HN),
        Wg1.T, bg1.reshape(1, DG), Wg2.T, bg2.reshape(1, DG),
        Wg3.T, bg3.reshape(1, DG))
    return (h_e, h_n, h_u)
